# baseline (speedup 1.0000x reference)
import functools

import jax
import jax.numpy as jnp
from jax import lax
from jax.experimental import pallas as pl
from jax.experimental.pallas import tpu as pltpu
from jax.experimental.pallas import tpu_sc as plsc

B, L, D_IN = 16, 256, 384
D = 4048
K = 512
N = B * L
BLK = 1024
NSTEPS = N // BLK

_DOT11 = (((1,), (1,)), ((), ()))


def _main_body(in_ref, wpre_ref, bpre_ref, cb_ref, wpost_ref, bpost_ref,
               idx_ref, loss_ref, wq_ref, csum_scr):
    @pl.when(pl.program_id(0) == 0)
    def _init():
        sq = cb_ref[...] * cb_ref[...]
        ones = jnp.ones((1, D), jnp.float32)
        csum_scr[...] = lax.dot_general(
            ones, sq, _DOT11, preferred_element_type=jnp.float32)
        loss_ref[...] = jnp.zeros_like(loss_ref)

    x = lax.dot_general(in_ref[...], wpre_ref[...], _DOT11,
                        preferred_element_type=jnp.float32) + bpre_ref[...]
    a = jnp.sum(x * x, axis=1, keepdims=True)
    dm = lax.dot_general(x, cb_ref[...], _DOT11,
                         preferred_element_type=jnp.float32)
    dist = (a - 2.0 * dm) + csum_scr[...]
    neg = -dist
    m = jnp.max(neg, axis=1, keepdims=True)
    ids = lax.broadcasted_iota(jnp.int32, dist.shape, 1)
    idx_ref[...] = jnp.min(jnp.where(neg == m, ids, K), axis=1)
    loss_ref[...] = loss_ref[...] + jnp.sum(-m)

    @pl.when(pl.program_id(0) == NSTEPS - 1)
    def _fin():
        loss_ref[...] = loss_ref[...] * (1.25 / (N * D))
        wq_ref[...] = lax.dot_general(cb_ref[...], wpost_ref[...], _DOT11,
                                      preferred_element_type=jnp.float32) + bpost_ref[...]


_main_call = pl.pallas_call(
    _main_body,
    grid=(NSTEPS,),
    in_specs=[
        pl.BlockSpec((BLK, D_IN), lambda i: (i, 0)),
        pl.BlockSpec((D, D_IN), lambda i: (0, 0)),
        pl.BlockSpec((1, D), lambda i: (0, 0)),
        pl.BlockSpec((K, D), lambda i: (0, 0)),
        pl.BlockSpec((D_IN, D), lambda i: (0, 0)),
        pl.BlockSpec((1, D_IN), lambda i: (0, 0)),
    ],
    out_specs=[
        pl.BlockSpec((BLK,), lambda i: (i,)),
        pl.BlockSpec((1, 1), lambda i: (0, 0)),
        pl.BlockSpec((K, D_IN), lambda i: (0, 0)),
    ],
    out_shape=[
        jax.ShapeDtypeStruct((N,), jnp.int32),
        jax.ShapeDtypeStruct((1, 1), jnp.float32),
        jax.ShapeDtypeStruct((K, D_IN), jnp.float32),
    ],
    scratch_shapes=[pltpu.VMEM((1, K), jnp.float32)],
)

_NC, _NS = 2, 16
_NW = _NC * _NS
_B_PER_W = N // _NW


@functools.partial(
    pl.kernel,
    mesh=plsc.VectorSubcoreMesh(core_axis_name="c", subcore_axis_name="s"),
    out_type=jax.ShapeDtypeStruct((N, D_IN), jnp.float32),
    scratch_types=[
        pltpu.VMEM((_B_PER_W,), jnp.int32),
        pltpu.VMEM((_B_PER_W, D_IN), jnp.float32),
        pltpu.SemaphoreType.DMA,
    ],
)
def _gather_call(table_hbm, idx_hbm, out_hbm, idx_v, rows_v, sem):
    wid = lax.axis_index("s") * _NC + lax.axis_index("c")
    base = wid * _B_PER_W
    pltpu.sync_copy(idx_hbm.at[pl.ds(base, _B_PER_W)], idx_v)
    pltpu.async_copy(table_hbm.at[idx_v], rows_v, sem).wait()
    pltpu.sync_copy(rows_v, out_hbm.at[pl.ds(base, _B_PER_W)])


def kernel(inputs, Wpre, bpre, Wpost, bpost, codebook):
    flat_in = inputs.reshape(N, D_IN)
    idx, loss, wq = _main_call(flat_in, Wpre, bpre.reshape(1, D), codebook,
                               Wpost, bpost.reshape(1, D_IN))
    out = _gather_call(wq, idx)
    return out.reshape(B, L, D_IN), loss.reshape(())


# speedup vs baseline: 1.2666x; 1.2666x over previous
import functools

import jax
import jax.numpy as jnp
from jax import lax
from jax.experimental import pallas as pl
from jax.experimental.pallas import tpu as pltpu
from jax.experimental.pallas import tpu_sc as plsc

B, L, D_IN = 16, 256, 384
D = 4048
K = 512
N = B * L
BLK = 512
NSTEPS = N // BLK

_DOT11 = (((1,), (1,)), ((), ()))
_DOT00 = (((0,), (0,)), ((), ()))


def _main_body(in_hbm, wpre_hbm, bpre_hbm, cbt_hbm, wpostt_hbm, bpost_hbm,
               idx_ref, loss_ref, wq_ref,
               in_s, wpre_s, bpre_s, cbt_s, wpostt_s, bpost_s, csum_s, sems):
    t = pl.program_id(0)

    def cp(src, dst, i):
        return pltpu.make_async_copy(src, dst, sems.at[i])

    @pl.when(t == 0)
    def _start_resident():
        cp(in_hbm.at[pl.ds(0, BLK)], in_s.at[0], 0).start()
        cp(wpre_hbm, wpre_s, 2).start()
        cp(bpre_hbm, bpre_s, 3).start()
        cp(cbt_hbm, cbt_s, 4).start()
        cp(wpostt_hbm, wpostt_s, 5).start()
        cp(bpost_hbm, bpost_s, 6).start()

    @pl.when(t + 1 < NSTEPS)
    def _prefetch_next():
        cp(in_hbm.at[pl.ds((t + 1) * BLK, BLK)], in_s.at[(t + 1) % 2],
           (t + 1) % 2).start()

    cp(in_hbm.at[pl.ds(t * BLK, BLK)], in_s.at[t % 2], t % 2).wait()

    @pl.when(t == 0)
    def _wait_pre():
        cp(wpre_hbm, wpre_s, 2).wait()
        cp(bpre_hbm, bpre_s, 3).wait()

    x = lax.dot_general(in_s[t % 2], wpre_s[...], _DOT11,
                        preferred_element_type=jnp.float32) + bpre_s[...][None, :]
    a = jnp.sum(x * x, axis=1, keepdims=True)

    @pl.when(t == 0)
    def _init():
        cp(cbt_hbm, cbt_s, 4).wait()
        sq = cbt_s[...] * cbt_s[...]
        ones = jnp.ones((1, D), jnp.float32)
        csum_s[...] = jnp.dot(ones, sq, preferred_element_type=jnp.float32)
        loss_ref[...] = jnp.zeros_like(loss_ref)

    dm = jnp.dot(x, cbt_s[...], preferred_element_type=jnp.float32)
    dist = (a - 2.0 * dm) + csum_s[...]
    neg = -dist
    m = jnp.max(neg, axis=1, keepdims=True)
    ids = lax.broadcasted_iota(jnp.int32, dist.shape, 1)
    idx_ref[...] = jnp.min(jnp.where(neg == m, ids, K), axis=1)
    loss_ref[...] = loss_ref[...] + jnp.sum(-m)

    @pl.when(t == NSTEPS - 1)
    def _fin():
        loss_ref[...] = loss_ref[...] * (1.25 / (N * D))
        cp(wpostt_hbm, wpostt_s, 5).wait()
        cp(bpost_hbm, bpost_s, 6).wait()
        wq_ref[...] = lax.dot_general(
            cbt_s[...], wpostt_s[...], _DOT00,
            preferred_element_type=jnp.float32) + bpost_s[...][None, :]


_main_call = pl.pallas_call(
    _main_body,
    grid=(NSTEPS,),
    in_specs=[
        pl.BlockSpec(memory_space=pl.ANY),
        pl.BlockSpec(memory_space=pl.ANY),
        pl.BlockSpec(memory_space=pl.ANY),
        pl.BlockSpec(memory_space=pl.ANY),
        pl.BlockSpec(memory_space=pl.ANY),
        pl.BlockSpec(memory_space=pl.ANY),
    ],
    out_specs=[
        pl.BlockSpec((BLK,), lambda i: (i,)),
        pl.BlockSpec((1, 1), lambda i: (0, 0)),
        pl.BlockSpec((K, D_IN), lambda i: (0, 0)),
    ],
    out_shape=[
        jax.ShapeDtypeStruct((N,), jnp.int32),
        jax.ShapeDtypeStruct((1, 1), jnp.float32),
        jax.ShapeDtypeStruct((K, D_IN), jnp.float32),
    ],
    scratch_shapes=[
        pltpu.VMEM((2, BLK, D_IN), jnp.float32),
        pltpu.VMEM((D, D_IN), jnp.float32),
        pltpu.VMEM((D,), jnp.float32),
        pltpu.VMEM((D, K), jnp.float32),
        pltpu.VMEM((D, D_IN), jnp.float32),
        pltpu.VMEM((D_IN,), jnp.float32),
        pltpu.VMEM((1, K), jnp.float32),
        pltpu.SemaphoreType.DMA((8,)),
    ],
)

_NC, _NS = 2, 16
_NW = _NC * _NS
_B_PER_W = N // _NW


@functools.partial(
    pl.kernel,
    mesh=plsc.VectorSubcoreMesh(core_axis_name="c", subcore_axis_name="s"),
    out_type=jax.ShapeDtypeStruct((N, D_IN), jnp.float32),
    scratch_types=[
        pltpu.VMEM((_B_PER_W,), jnp.int32),
        pltpu.VMEM((_B_PER_W, D_IN), jnp.float32),
        pltpu.SemaphoreType.DMA,
    ],
)
def _gather_call(table_hbm, idx_hbm, out_hbm, idx_v, rows_v, sem):
    wid = lax.axis_index("s") * _NC + lax.axis_index("c")
    base = wid * _B_PER_W
    pltpu.sync_copy(idx_hbm.at[pl.ds(base, _B_PER_W)], idx_v)
    pltpu.async_copy(table_hbm.at[idx_v], rows_v, sem).wait()
    pltpu.sync_copy(rows_v, out_hbm.at[pl.ds(base, _B_PER_W)])


def kernel(inputs, Wpre, bpre, Wpost, bpost, codebook):
    flat_in = inputs.reshape(N, D_IN)
    idx, loss, wq = _main_call(flat_in, Wpre, bpre, codebook.T, Wpost.T, bpost)
    out = _gather_call(wq, idx)
    return out.reshape(B, L, D_IN), loss.reshape(())
